# Initial kernel scaffold; baseline (speedup 1.0000x reference)
import jax
import jax.numpy as jnp
from jax.experimental import pallas as pl


def kernel(transmissions, susceptibilities, beta_parameters, people_school, groups_school, people_company, groups_company, people_household, groups_household):
    raise NotImplementedError("write your pallas kernel here")


# speedup vs baseline: 154.9895x; 154.9895x over previous
import functools

import jax
import jax.numpy as jnp
import numpy as np
from jax import lax
from jax.experimental import pallas as pl
from jax.experimental.pallas import tpu as pltpu
from jax.experimental.pallas import tpu_sc as plsc

N = 100000
G = 10000
E = 1600000
T = 5
TAU = 0.1
NW = 32
NC = 2
EW = E // NW
CHUNK = 10000
NCHUNK = EW // CHUNK
NPAD = 100352
PW = NPAD // NW
GPAD = 10240
GSUMW = 3 * GPAD // NW
PBITS = 17

_MESH = plsc.VectorSubcoreMesh(core_axis_name="c", subcore_axis_name="s")
_PARAMS = pltpu.CompilerParams(needs_layout_passes=False)
_F32 = jnp.float32
_I32 = jnp.int32


def _wid():
    return lax.axis_index("s") * NC + lax.axis_index("c")


def _zero_ref(ref, nwords):
    z = jnp.zeros((16,), _F32)

    def body(i, _):
        ref[pl.ds(i * 16, 16)] = z
        return 0

    lax.fori_loop(0, nwords // 16, body, 0)


@functools.partial(
    pl.kernel,
    out_type=jax.ShapeDtypeStruct((NW * 3 * GPAD,), _F32),
    mesh=_MESH,
    compiler_params=_PARAMS,
    scratch_types=[
        pltpu.VMEM((NPAD,), _F32),
        pltpu.VMEM((GPAD,), _F32),
        pltpu.VMEM((CHUNK,), _I32),
    ],
)
def _p1(trans_hbm, edges_hbm, gparts_hbm, trans_v, gpart_v, ebuf_v):
    w = _wid()
    pltpu.sync_copy(trans_hbm, trans_v)
    for a in range(3):
        _zero_ref(gpart_v, GPAD)
        for c in range(NCHUNK):
            off = (a * NW + w) * EW + c * CHUNK
            pltpu.sync_copy(edges_hbm.at[pl.ds(off, CHUNK)], ebuf_v)

            def body(i, _):
                packed = ebuf_v[pl.ds(i * 16, 16)]
                p = jnp.bitwise_and(packed, (1 << PBITS) - 1)
                g = jnp.right_shift(packed, PBITS)
                tv = plsc.load_gather(trans_v, [p])
                plsc.addupdate_scatter(gpart_v, [g], tv)
                return 0

            lax.fori_loop(0, CHUNK // 16, body, 0)
        pltpu.sync_copy(gpart_v, gparts_hbm.at[pl.ds((w * 3 + a) * GPAD, GPAD)])


@functools.partial(
    pl.kernel,
    out_type=jax.ShapeDtypeStruct((3 * GPAD,), _F32),
    mesh=_MESH,
    compiler_params=_PARAMS,
    scratch_types=[
        pltpu.VMEM((GSUMW,), _F32),
        pltpu.VMEM((GSUMW,), _F32),
    ],
)
def _p1b(gparts_hbm, gsum_hbm, acc_v, buf_v):
    w = _wid()
    _zero_ref(acc_v, GSUMW)

    def outer(r, _):
        pltpu.sync_copy(gparts_hbm.at[pl.ds(r * 3 * GPAD + w * GSUMW, GSUMW)], buf_v)

        def inner(i, _):
            sl = pl.ds(i * 16, 16)
            acc_v[sl] = acc_v[sl] + buf_v[sl]
            return 0

        lax.fori_loop(0, GSUMW // 16, inner, 0)
        return 0

    lax.fori_loop(0, NW, outer, 0)
    pltpu.sync_copy(acc_v, gsum_hbm.at[pl.ds(w * GSUMW, GSUMW)])


@functools.partial(
    pl.kernel,
    out_type=jax.ShapeDtypeStruct((NW * NPAD,), _F32),
    mesh=_MESH,
    compiler_params=_PARAMS,
    scratch_types=[
        pltpu.VMEM((NPAD,), _F32),
        pltpu.VMEM((GPAD,), _F32),
        pltpu.VMEM((CHUNK,), _I32),
        pltpu.VMEM((16,), _F32),
    ],
)
def _p2(edges_hbm, gsum_hbm, beta_hbm, eparts_hbm, epart_v, gbuf_v, ebuf_v, beta_v):
    w = _wid()
    pltpu.sync_copy(beta_hbm, beta_v)
    _zero_ref(epart_v, NPAD)
    for a in range(3):
        pltpu.sync_copy(gsum_hbm.at[pl.ds(a * GPAD, GPAD)], gbuf_v)
        beta_a = plsc.load_gather(beta_v, [jnp.full((16,), a, _I32)])
        for c in range(NCHUNK):
            off = (a * NW + w) * EW + c * CHUNK
            pltpu.sync_copy(edges_hbm.at[pl.ds(off, CHUNK)], ebuf_v)

            def body(i, _):
                packed = ebuf_v[pl.ds(i * 16, 16)]
                p = jnp.bitwise_and(packed, (1 << PBITS) - 1)
                g = jnp.right_shift(packed, PBITS)
                gv = plsc.load_gather(gbuf_v, [g])
                plsc.addupdate_scatter(epart_v, [p], gv * beta_a)
                return 0

            lax.fori_loop(0, CHUNK // 16, body, 0)
    pltpu.sync_copy(epart_v, eparts_hbm.at[pl.ds(w * NPAD, NPAD)])


@functools.partial(
    pl.kernel,
    out_type=jax.ShapeDtypeStruct((NPAD,), _F32),
    mesh=_MESH,
    compiler_params=_PARAMS,
    scratch_types=[
        pltpu.VMEM((PW,), _F32),
        pltpu.VMEM((PW,), _F32),
    ],
)
def _p3w(parts_hbm, out_hbm, acc_v, buf_v):
    w = _wid()
    _zero_ref(acc_v, PW)

    def outer(r, _):
        pltpu.sync_copy(parts_hbm.at[pl.ds(r * NPAD + w * PW, PW)], buf_v)

        def inner(i, _):
            sl = pl.ds(i * 16, 16)
            acc_v[sl] = acc_v[sl] + buf_v[sl]
            return 0

        lax.fori_loop(0, PW // 16, inner, 0)
        return 0

    lax.fori_loop(0, NW, outer, 0)
    pltpu.sync_copy(acc_v, out_hbm.at[pl.ds(w * PW, PW)])


def _p3_body(t):
    @functools.partial(
        pl.kernel,
        out_type=(
            jax.ShapeDtypeStruct((NPAD,), _F32),
            jax.ShapeDtypeStruct((NPAD,), _F32),
            jax.ShapeDtypeStruct((NPAD,), _F32),
        ),
        mesh=_MESH,
        compiler_params=_PARAMS,
        scratch_types=[
            pltpu.VMEM((PW,), _F32),
            pltpu.VMEM((PW,), _F32),
            pltpu.VMEM((PW,), _F32),
            pltpu.VMEM((PW,), _F32),
            pltpu.VMEM((PW,), _F32),
            pltpu.VMEM((PW,), _F32),
            pltpu.VMEM((PW,), _F32),
        ],
    )
    def _p3(eparts_hbm, wdeg_hbm, trans_hbm, susc_hbm, noise_hbm,
            ni_hbm, trans2_hbm, susc2_hbm,
            acc_v, buf_v, wdeg_v, tr_v, su_v, no_v, ni_v):
        w = _wid()
        base = w * PW
        _zero_ref(acc_v, PW)

        def outer(r, _):
            pltpu.sync_copy(eparts_hbm.at[pl.ds(r * NPAD + base, PW)], buf_v)

            def inner(i, _):
                sl = pl.ds(i * 16, 16)
                acc_v[sl] = acc_v[sl] + buf_v[sl]
                return 0

            lax.fori_loop(0, PW // 16, inner, 0)
            return 0

        lax.fori_loop(0, NW, outer, 0)
        pltpu.sync_copy(wdeg_hbm.at[pl.ds(base, PW)], wdeg_v)
        pltpu.sync_copy(trans_hbm.at[pl.ds(base, PW)], tr_v)
        pltpu.sync_copy(susc_hbm.at[pl.ds(base, PW)], su_v)
        pltpu.sync_copy(noise_hbm.at[pl.ds(t * NPAD + base, PW)], no_v)

        def tail(i, _):
            sl = pl.ds(i * 16, 16)
            expo = acc_v[sl] - wdeg_v[sl] * tr_v[sl]
            x = su_v[sl] * expo
            p = 1.0 - jnp.exp(-x)
            p = jnp.clip(p, np.float32(1e-9), np.float32(1.0 - 1e-9))
            q = 1.0 - p
            s = (q / p) * no_v[sl]
            s2 = s * s
            s4 = s2 * s2
            s8 = s4 * s4
            a10 = s8 * s2
            ni = 1.0 / (1.0 + a10)
            ni_v[sl] = ni
            tr_v[sl] = tr_v[sl] + 0.2 * ni
            su_v[sl] = su_v[sl] - ni
            return 0

        lax.fori_loop(0, PW // 16, tail, 0)
        pltpu.sync_copy(ni_v, ni_hbm.at[pl.ds(base, PW)])
        pltpu.sync_copy(tr_v, trans2_hbm.at[pl.ds(base, PW)])
        pltpu.sync_copy(su_v, susc2_hbm.at[pl.ds(base, PW)])

    return _p3


_P3 = [_p3_body(t) for t in range(T)]


def kernel(transmissions, susceptibilities, beta_parameters,
           people_school, groups_school,
           people_company, groups_company,
           people_household, groups_household):
    def pack(p, g):
        return jnp.bitwise_or(p.astype(_I32),
                              jnp.left_shift(g.astype(_I32), PBITS))

    edges = jnp.concatenate([
        pack(people_school, groups_school),
        pack(people_company, groups_company),
        pack(people_household, groups_household),
    ])

    trans0 = jnp.pad(transmissions.astype(_F32), (0, NPAD - N))
    susc0 = jnp.pad(susceptibilities.astype(_F32), (0, NPAD - N))
    beta = jnp.pad(beta_parameters.astype(_F32), (0, 16 - 3))

    noise_key = jax.random.key(42)
    wn = []
    for t in range(T):
        u = jax.random.uniform(jax.random.fold_in(noise_key, t), (2, N),
                               minval=1e-9, maxval=1.0)
        wt = jnp.log(u[0]) / jnp.log(u[1])
        wn.append(jnp.pad(wt, (0, NPAD - N), constant_values=1.0))
    noise = jnp.concatenate(wn)

    ones_gsum = jnp.ones((3 * GPAD,), _F32)
    wparts = _p2(edges, ones_gsum, beta)
    wdeg = _p3w(wparts)

    trans, susc = trans0, susc0
    rets = []
    for t in range(T):
        gparts = _p1(trans, edges)
        gsum = _p1b(gparts)
        eparts = _p2(edges, gsum, beta)
        ni, trans, susc = _P3[t](eparts, wdeg, trans, susc, noise)
        rets.append(ni[:N])
    return jnp.stack(rets, axis=0)


# speedup vs baseline: 223.7883x; 1.4439x over previous
import functools

import jax
import jax.numpy as jnp
import numpy as np
from jax import lax
from jax.experimental import pallas as pl
from jax.experimental.pallas import tpu as pltpu
from jax.experimental.pallas import tpu_sc as plsc

N = 100000
G = 10000
E = 1600000
T = 5
NW = 32
NC = 2
EW = E // NW
CHUNK = 10000
NCHUNK = EW // CHUNK
NPAD = 100352
PW = NPAD // NW
GPAD = 10240
GSUMW = 3 * GPAD // NW
PBITS = 17
PMASK = (1 << PBITS) - 1
UE = 25

_MESH = plsc.VectorSubcoreMesh(core_axis_name="c", subcore_axis_name="s")
_PARAMS = pltpu.CompilerParams(needs_layout_passes=False)
_F32 = jnp.float32
_I32 = jnp.int32


def _wid():
    return lax.axis_index("s") * NC + lax.axis_index("c")


def _zero_ref(ref, nwords, unroll=16):
    z = jnp.zeros((16,), _F32)
    nv = nwords // 16

    def body(j, _):
        for u in range(unroll):
            ref[pl.ds((j * unroll + u) * 16, 16)] = z
        return 0

    lax.fori_loop(0, nv // unroll, body, 0)
    for r in range(nv - (nv // unroll) * unroll):
        ref[pl.ds(((nv // unroll) * unroll + r) * 16, 16)] = z


def _edge_stream(edges_hbm, w, ebufs, sems, per_activity, at_boundary):
    def issue(a, c, b):
        off = (a * NW + w) * EW + c * CHUNK
        return pltpu.async_copy(edges_hbm.at[pl.ds(off, CHUNK)], ebufs[b], sems[b])

    nxt = issue(0, 0, 0)
    for a in range(3):
        inner = per_activity(a)
        d = [None] * NCHUNK
        d[0] = nxt
        for c in range(NCHUNK):
            b = (a * NCHUNK + c) % 2
            if c + 1 < NCHUNK:
                d[c + 1] = issue(a, c + 1, 1 - b)
            elif a + 1 < 3:
                nxt = issue(a + 1, 0, 1 - b)
            d[c].wait()
            ebuf = ebufs[b]

            def body(j, _, inner=inner, ebuf=ebuf):
                for u in range(UE):
                    inner(ebuf, j * UE + u)
                return 0

            lax.fori_loop(0, (CHUNK // 16) // UE, body, 0)
        at_boundary(a)


@functools.partial(
    pl.kernel,
    out_type=jax.ShapeDtypeStruct((NW * 3 * GPAD,), _F32),
    mesh=_MESH,
    compiler_params=_PARAMS,
    scratch_types=[
        pltpu.VMEM((NPAD,), _F32),
        pltpu.VMEM((GPAD,), _F32),
        pltpu.VMEM((CHUNK,), _I32),
        pltpu.VMEM((CHUNK,), _I32),
        pltpu.SemaphoreType.DMA,
        pltpu.SemaphoreType.DMA,
        pltpu.SemaphoreType.DMA,
    ],
)
def _p1(trans_hbm, edges_hbm, gparts_hbm, trans_v, gpart_v, eb0, eb1,
        se0, se1, st):
    w = _wid()
    tcopy = pltpu.async_copy(trans_hbm, trans_v, st)
    _zero_ref(gpart_v, GPAD)
    tcopy.wait()

    def per_activity(a):
        def inner(ebuf, jv):
            packed = ebuf[pl.ds(jv * 16, 16)]
            p = jnp.bitwise_and(packed, PMASK)
            g = jnp.right_shift(packed, PBITS)
            tv = plsc.load_gather(trans_v, [p])
            plsc.addupdate_scatter(gpart_v, [g], tv)
        return inner

    def at_boundary(a):
        pltpu.sync_copy(gpart_v, gparts_hbm.at[pl.ds((w * 3 + a) * GPAD, GPAD)])
        if a + 1 < 3:
            _zero_ref(gpart_v, GPAD)

    _edge_stream(edges_hbm, w, (eb0, eb1), (se0, se1), per_activity, at_boundary)


@functools.partial(
    pl.kernel,
    out_type=jax.ShapeDtypeStruct((3 * GPAD,), _F32),
    mesh=_MESH,
    compiler_params=_PARAMS,
    scratch_types=[
        pltpu.VMEM((NW * GSUMW,), _F32),
        pltpu.VMEM((GSUMW,), _F32),
        pltpu.SemaphoreType.DMA,
    ],
)
def _p1b(gparts_hbm, gsum_hbm, rows_v, acc_v, sem):
    w = _wid()
    descs = [
        pltpu.async_copy(
            gparts_hbm.at[pl.ds(r * 3 * GPAD + w * GSUMW, GSUMW)],
            rows_v.at[pl.ds(r * GSUMW, GSUMW)], sem)
        for r in range(NW)
    ]
    for d in descs:
        d.wait()
    nv = GSUMW // 16

    def body(j, _):
        for u in range(4):
            i = j * 4 + u
            s = rows_v[pl.ds(i * 16, 16)]
            for r in range(1, NW):
                s = s + rows_v[pl.ds(r * GSUMW + i * 16, 16)]
            acc_v[pl.ds(i * 16, 16)] = s
        return 0

    lax.fori_loop(0, nv // 4, body, 0)
    pltpu.sync_copy(acc_v, gsum_hbm.at[pl.ds(w * GSUMW, GSUMW)])


@functools.partial(
    pl.kernel,
    out_type=jax.ShapeDtypeStruct((NW * NPAD,), _F32),
    mesh=_MESH,
    compiler_params=_PARAMS,
    scratch_types=[
        pltpu.VMEM((NPAD,), _F32),
        pltpu.VMEM((GPAD,), _F32),
        pltpu.VMEM((CHUNK,), _I32),
        pltpu.VMEM((CHUNK,), _I32),
        pltpu.VMEM((16,), _F32),
        pltpu.SemaphoreType.DMA,
        pltpu.SemaphoreType.DMA,
        pltpu.SemaphoreType.DMA,
    ],
)
def _p2(edges_hbm, gsum_hbm, beta_hbm, eparts_hbm, epart_v, gbuf_v,
        eb0, eb1, beta_v, se0, se1, sg):
    w = _wid()
    pltpu.sync_copy(beta_hbm, beta_v)
    gd = pltpu.async_copy(gsum_hbm.at[pl.ds(0, GPAD)], gbuf_v, sg)
    _zero_ref(epart_v, NPAD)

    state = {"gd": gd}

    def per_activity(a):
        state["gd"].wait()
        beta_a = plsc.load_gather(beta_v, [jnp.full((16,), a, _I32)])

        def inner(ebuf, jv):
            packed = ebuf[pl.ds(jv * 16, 16)]
            p = jnp.bitwise_and(packed, PMASK)
            g = jnp.right_shift(packed, PBITS)
            gv = plsc.load_gather(gbuf_v, [g])
            plsc.addupdate_scatter(epart_v, [p], gv * beta_a)
        return inner

    def at_boundary(a):
        if a + 1 < 3:
            state["gd"] = pltpu.async_copy(
                gsum_hbm.at[pl.ds((a + 1) * GPAD, GPAD)], gbuf_v, sg)

    _edge_stream(edges_hbm, w, (eb0, eb1), (se0, se1), per_activity, at_boundary)
    pltpu.sync_copy(epart_v, eparts_hbm.at[pl.ds(w * NPAD, NPAD)])


@functools.partial(
    pl.kernel,
    out_type=jax.ShapeDtypeStruct((NPAD,), _F32),
    mesh=_MESH,
    compiler_params=_PARAMS,
    scratch_types=[
        pltpu.VMEM((NW * PW,), _F32),
        pltpu.VMEM((PW,), _F32),
        pltpu.SemaphoreType.DMA,
    ],
)
def _p3w(parts_hbm, out_hbm, rows_v, acc_v, sem):
    w = _wid()
    base = w * PW
    descs = [
        pltpu.async_copy(parts_hbm.at[pl.ds(r * NPAD + base, PW)],
                         rows_v.at[pl.ds(r * PW, PW)], sem)
        for r in range(NW)
    ]
    for d in descs:
        d.wait()
    nv = PW // 16

    def body(j, _):
        for u in range(4):
            i = j * 4 + u
            s = rows_v[pl.ds(i * 16, 16)]
            for r in range(1, NW):
                s = s + rows_v[pl.ds(r * PW + i * 16, 16)]
            acc_v[pl.ds(i * 16, 16)] = s
        return 0

    lax.fori_loop(0, nv // 4, body, 0)
    pltpu.sync_copy(acc_v, out_hbm.at[pl.ds(base, PW)])


def _p3_body(t):
    @functools.partial(
        pl.kernel,
        out_type=(
            jax.ShapeDtypeStruct((NPAD,), _F32),
            jax.ShapeDtypeStruct((NPAD,), _F32),
            jax.ShapeDtypeStruct((NPAD,), _F32),
        ),
        mesh=_MESH,
        compiler_params=_PARAMS,
        scratch_types=[
            pltpu.VMEM((NW * PW,), _F32),
            pltpu.VMEM((PW,), _F32),
            pltpu.VMEM((PW,), _F32),
            pltpu.VMEM((PW,), _F32),
            pltpu.VMEM((PW,), _F32),
            pltpu.VMEM((PW,), _F32),
            pltpu.SemaphoreType.DMA,
        ],
    )
    def _p3(eparts_hbm, wdeg_hbm, trans_hbm, susc_hbm, noise_hbm,
            ni_hbm, trans2_hbm, susc2_hbm,
            rows_v, wdeg_v, tr_v, su_v, no_v, ni_v, sem):
        w = _wid()
        base = w * PW
        descs = [
            pltpu.async_copy(eparts_hbm.at[pl.ds(r * NPAD + base, PW)],
                             rows_v.at[pl.ds(r * PW, PW)], sem)
            for r in range(NW)
        ]
        descs.append(pltpu.async_copy(wdeg_hbm.at[pl.ds(base, PW)], wdeg_v, sem))
        descs.append(pltpu.async_copy(trans_hbm.at[pl.ds(base, PW)], tr_v, sem))
        descs.append(pltpu.async_copy(susc_hbm.at[pl.ds(base, PW)], su_v, sem))
        descs.append(pltpu.async_copy(
            noise_hbm.at[pl.ds(t * NPAD + base, PW)], no_v, sem))
        for d in descs:
            d.wait()
        nv = PW // 16

        def body(j, _):
            for u in range(4):
                i = j * 4 + u
                sl = pl.ds(i * 16, 16)
                s = rows_v[sl]
                for r in range(1, NW):
                    s = s + rows_v[pl.ds(r * PW + i * 16, 16)]
                expo = s - wdeg_v[sl] * tr_v[sl]
                x = su_v[sl] * expo
                p = 1.0 - jnp.exp(-x)
                p = jnp.clip(p, np.float32(1e-9), np.float32(1.0 - 1e-9))
                q = 1.0 - p
                sg = (q / p) * no_v[sl]
                s2 = sg * sg
                s4 = s2 * s2
                s8 = s4 * s4
                a10 = s8 * s2
                ni = 1.0 / (1.0 + a10)
                ni_v[sl] = ni
                tr_v[sl] = tr_v[sl] + 0.2 * ni
                su_v[sl] = su_v[sl] - ni
            return 0

        lax.fori_loop(0, nv // 4, body, 0)
        pltpu.sync_copy(ni_v, ni_hbm.at[pl.ds(base, PW)])
        pltpu.sync_copy(tr_v, trans2_hbm.at[pl.ds(base, PW)])
        pltpu.sync_copy(su_v, susc2_hbm.at[pl.ds(base, PW)])

    return _p3


_P3 = [_p3_body(t) for t in range(T)]


def kernel(transmissions, susceptibilities, beta_parameters,
           people_school, groups_school,
           people_company, groups_company,
           people_household, groups_household):
    def pack(p, g):
        return jnp.bitwise_or(p.astype(_I32),
                              jnp.left_shift(g.astype(_I32), PBITS))

    edges = jnp.concatenate([
        pack(people_school, groups_school),
        pack(people_company, groups_company),
        pack(people_household, groups_household),
    ])

    trans0 = jnp.pad(transmissions.astype(_F32), (0, NPAD - N))
    susc0 = jnp.pad(susceptibilities.astype(_F32), (0, NPAD - N))
    beta = jnp.pad(beta_parameters.astype(_F32), (0, 16 - 3))

    noise_key = jax.random.key(42)
    wn = []
    for t in range(T):
        u = jax.random.uniform(jax.random.fold_in(noise_key, t), (2, N),
                               minval=1e-9, maxval=1.0)
        wt = jnp.log(u[0]) / jnp.log(u[1])
        wn.append(jnp.pad(wt, (0, NPAD - N), constant_values=1.0))
    noise = jnp.concatenate(wn)

    ones_gsum = jnp.ones((3 * GPAD,), _F32)
    wparts = _p2(edges, ones_gsum, beta)
    wdeg = _p3w(wparts)

    trans, susc = trans0, susc0
    rets = []
    for t in range(T):
        gparts = _p1(trans, edges)
        gsum = _p1b(gparts)
        eparts = _p2(edges, gsum, beta)
        ni, trans, susc = _P3[t](eparts, wdeg, trans, susc, noise)
        rets.append(ni[:N])
    return jnp.stack(rets, axis=0)


# speedup vs baseline: 240.1200x; 1.0730x over previous
import functools

import jax
import jax.numpy as jnp
import numpy as np
from jax import lax
from jax.experimental import pallas as pl
from jax.experimental.pallas import tpu as pltpu
from jax.experimental.pallas import tpu_sc as plsc

N = 100000
G = 10000
E = 1600000
T = 5
NW = 32
NC = 2
NS = 16
EW = E // NW
CHUNK = 2000
NCHUNK = EW // CHUNK
NPAD = 100352
PW = NPAD // NW
GPAD = 10240
NROWS = NW + 2
GSUMW = 3 * GPAD // NW
PBITS = 17
PMASK = (1 << PBITS) - 1
NV = CHUNK // 16
SV = 50
TECV = NV - SV
SE = SV * 16
UT = 5
US = 5

_MESH = plsc.VectorSubcoreMesh(core_axis_name="c", subcore_axis_name="s")
_PARAMS = pltpu.CompilerParams(needs_layout_passes=False)
_F32 = jnp.float32
_I32 = jnp.int32


def _wid():
    return lax.axis_index("s") * NC + lax.axis_index("c")


def _zero_ref(ref, nwords, unroll=16):
    z = jnp.zeros((16,), _F32)
    nv = nwords // 16

    def body(j, _):
        for u in range(unroll):
            ref[pl.ds((j * unroll + u) * 16, 16)] = z
        return 0

    lax.fori_loop(0, nv // unroll, body, 0)
    for r in range(nv - (nv // unroll) * unroll):
        ref[pl.ds(((nv // unroll) * unroll + r) * 16, 16)] = z


def _edge_engine(edges_hbm, w, ebufs, esems, gxs, vls, ssems, spacc,
                 per_activity, at_boundary):
    def eoff(a, c):
        return (a * NW + w) * EW + c * CHUNK

    def issue_e(a, c, b):
        return pltpu.async_copy(
            edges_hbm.at[pl.ds(eoff(a, c), CHUNK)], ebufs[b], esems[b])

    def wait_e(a, c, b):
        pltpu.make_async_copy(
            edges_hbm.at[pl.ds(eoff(a, c), CHUNK)], ebufs[b], esems[b]).wait()

    def issue_s(sl):
        pltpu.async_copy(vls[sl], spacc.at[gxs[sl]], ssems[sl], add=True)

    def wait_s(sl):
        pltpu.make_async_copy(vls[sl], spacc.at[gxs[sl]], ssems[sl]).wait()

    def proc(ebuf, tec_fn, str_fn, slot):
        def bodyT(i, _):
            for u in range(UT):
                tec_fn(ebuf, i * UT + u)
            return 0

        lax.fori_loop(0, TECV // UT, bodyT, 0)
        if slot is None:
            def bodyS0(i, _):
                for u in range(US):
                    tec_fn(ebuf, TECV + i * US + u)
                return 0

            lax.fori_loop(0, SV // US, bodyS0, 0)
        else:
            def bodyS(i, _):
                for u in range(US):
                    so = i * US + u
                    iv, vv = str_fn(ebuf, TECV + so)
                    gxs[slot][pl.ds(so * 16, 16)] = iv
                    vls[slot][pl.ds(so * 16, 16)] = vv
                return 0

            lax.fori_loop(0, SV // US, bodyS, 0)
            issue_s(slot)

    for a in range(3):
        tec_fn, str_fn = per_activity(a)
        issue_e(a, 0, 0)
        issue_e(a, 1, 1)
        wait_e(a, 0, 0)
        proc(ebufs[0], tec_fn, str_fn, None)
        issue_e(a, 2, 0)

        def pair(j, _):
            @pl.when(j >= 1)
            def _():
                wait_s(1)
                wait_s(0)
            c1 = 2 * j + 1
            wait_e(a, c1, 1)
            proc(ebufs[1], tec_fn, str_fn, 1)
            issue_e(a, c1 + 2, 1)
            c2 = 2 * j + 2
            wait_e(a, c2, 0)
            proc(ebufs[0], tec_fn, str_fn, 0)
            issue_e(a, c2 + 2, 0)
            return 0

        lax.fori_loop(0, (NCHUNK - 3) // 2, pair, 0)
        wait_s(1)
        wait_s(0)
        wait_e(a, NCHUNK - 2, 1)
        proc(ebufs[1], tec_fn, str_fn, 1)
        wait_e(a, NCHUNK - 1, 0)
        proc(ebufs[0], tec_fn, str_fn, 0)
        wait_s(1)
        wait_s(0)
        at_boundary(a)


@functools.partial(
    pl.kernel,
    out_type=jax.ShapeDtypeStruct((NROWS * 3 * GPAD,), _F32),
    mesh=_MESH,
    compiler_params=_PARAMS,
    scratch_types=[
        pltpu.VMEM((NPAD,), _F32),
        pltpu.VMEM((GPAD,), _F32),
        pltpu.VMEM((CHUNK,), _I32),
        pltpu.VMEM((CHUNK,), _I32),
        pltpu.VMEM((SE,), _I32),
        pltpu.VMEM((SE,), _I32),
        pltpu.VMEM((SE,), _F32),
        pltpu.VMEM((SE,), _F32),
        pltpu.VMEM_SHARED((3 * GPAD,), _F32),
        pltpu.SemaphoreType.DMA,
        pltpu.SemaphoreType.DMA,
        pltpu.SemaphoreType.DMA,
        pltpu.SemaphoreType.DMA,
        pltpu.SemaphoreType.DMA,
    ],
)
def _p1(trans_hbm, edges_hbm, gparts_hbm, trans_v, gpart_v, eb0, eb1,
        gxa, gxb, vla, vlb, spacc, se0, se1, ss0, ss1, st):
    w = _wid()
    sid = lax.axis_index("s")
    cid = lax.axis_index("c")
    tcopy = pltpu.async_copy(trans_hbm, trans_v, st)
    _zero_ref(gpart_v, GPAD)

    @pl.when(sid == 0)
    def _():
        for a3 in range(3):
            pltpu.sync_copy(gpart_v, spacc.at[pl.ds(a3 * GPAD, GPAD)])

    plsc.subcore_barrier()
    tcopy.wait()

    def per_activity(a):
        goff = a * GPAD

        def tec_fn(ebuf, jv):
            packed = ebuf[pl.ds(jv * 16, 16)]
            p = jnp.bitwise_and(packed, PMASK)
            g = jnp.right_shift(packed, PBITS)
            tv = plsc.load_gather(trans_v, [p])
            plsc.addupdate_scatter(gpart_v, [g], tv)

        def str_fn(ebuf, jv):
            packed = ebuf[pl.ds(jv * 16, 16)]
            p = jnp.bitwise_and(packed, PMASK)
            g = jnp.right_shift(packed, PBITS)
            tv = plsc.load_gather(trans_v, [p])
            return g + goff, tv

        return tec_fn, str_fn

    def at_boundary(a):
        pltpu.sync_copy(gpart_v, gparts_hbm.at[pl.ds((w * 3 + a) * GPAD, GPAD)])
        if a + 1 < 3:
            _zero_ref(gpart_v, GPAD)

    _edge_engine(edges_hbm, w, (eb0, eb1), (se0, se1), (gxa, gxb),
                 (vla, vlb), (ss0, ss1), spacc, per_activity, at_boundary)
    plsc.subcore_barrier()
    sw = 3 * GPAD // NS
    pltpu.sync_copy(
        spacc.at[pl.ds(sid * sw, sw)],
        gparts_hbm.at[pl.ds((NW + cid) * 3 * GPAD + sid * sw, sw)])


@functools.partial(
    pl.kernel,
    out_type=jax.ShapeDtypeStruct((3 * GPAD,), _F32),
    mesh=_MESH,
    compiler_params=_PARAMS,
    scratch_types=[
        pltpu.VMEM((NROWS * GSUMW,), _F32),
        pltpu.VMEM((GSUMW,), _F32),
        pltpu.SemaphoreType.DMA,
    ],
)
def _p1b(gparts_hbm, gsum_hbm, rows_v, acc_v, sem):
    w = _wid()
    descs = [
        pltpu.async_copy(
            gparts_hbm.at[pl.ds(r * 3 * GPAD + w * GSUMW, GSUMW)],
            rows_v.at[pl.ds(r * GSUMW, GSUMW)], sem)
        for r in range(NROWS)
    ]
    for d in descs:
        d.wait()
    nv = GSUMW // 16

    def body(j, _):
        for u in range(4):
            i = j * 4 + u
            s = rows_v[pl.ds(i * 16, 16)]
            for r in range(1, NROWS):
                s = s + rows_v[pl.ds(r * GSUMW + i * 16, 16)]
            acc_v[pl.ds(i * 16, 16)] = s
        return 0

    lax.fori_loop(0, nv // 4, body, 0)
    pltpu.sync_copy(acc_v, gsum_hbm.at[pl.ds(w * GSUMW, GSUMW)])


@functools.partial(
    pl.kernel,
    out_type=jax.ShapeDtypeStruct((NROWS * NPAD,), _F32),
    mesh=_MESH,
    compiler_params=_PARAMS,
    scratch_types=[
        pltpu.VMEM((NPAD,), _F32),
        pltpu.VMEM((GPAD,), _F32),
        pltpu.VMEM((CHUNK,), _I32),
        pltpu.VMEM((CHUNK,), _I32),
        pltpu.VMEM((SE,), _I32),
        pltpu.VMEM((SE,), _I32),
        pltpu.VMEM((SE,), _F32),
        pltpu.VMEM((SE,), _F32),
        pltpu.VMEM((16,), _F32),
        pltpu.VMEM_SHARED((NPAD,), _F32),
        pltpu.SemaphoreType.DMA,
        pltpu.SemaphoreType.DMA,
        pltpu.SemaphoreType.DMA,
        pltpu.SemaphoreType.DMA,
        pltpu.SemaphoreType.DMA,
    ],
)
def _p2(edges_hbm, gsum_hbm, beta_hbm, eparts_hbm, epart_v, gbuf_v,
        eb0, eb1, gxa, gxb, vla, vlb, beta_v, spacc, se0, se1, ss0, ss1, sg):
    w = _wid()
    sid = lax.axis_index("s")
    cid = lax.axis_index("c")
    pltpu.sync_copy(beta_hbm, beta_v)
    gd = pltpu.async_copy(gsum_hbm.at[pl.ds(0, GPAD)], gbuf_v, sg)
    _zero_ref(epart_v, NPAD)
    zw = NPAD // NS
    pltpu.sync_copy(epart_v.at[pl.ds(sid * zw, zw)],
                    spacc.at[pl.ds(sid * zw, zw)])
    plsc.subcore_barrier()

    state = {"gd": gd}

    def per_activity(a):
        state["gd"].wait()
        beta_a = plsc.load_gather(beta_v, [jnp.full((16,), a, _I32)])

        def tec_fn(ebuf, jv):
            packed = ebuf[pl.ds(jv * 16, 16)]
            p = jnp.bitwise_and(packed, PMASK)
            g = jnp.right_shift(packed, PBITS)
            gv = plsc.load_gather(gbuf_v, [g])
            plsc.addupdate_scatter(epart_v, [p], gv * beta_a)

        def str_fn(ebuf, jv):
            packed = ebuf[pl.ds(jv * 16, 16)]
            p = jnp.bitwise_and(packed, PMASK)
            g = jnp.right_shift(packed, PBITS)
            gv = plsc.load_gather(gbuf_v, [g])
            return p, gv * beta_a

        return tec_fn, str_fn

    def at_boundary(a):
        if a + 1 < 3:
            state["gd"] = pltpu.async_copy(
                gsum_hbm.at[pl.ds((a + 1) * GPAD, GPAD)], gbuf_v, sg)

    _edge_engine(edges_hbm, w, (eb0, eb1), (se0, se1), (gxa, gxb),
                 (vla, vlb), (ss0, ss1), spacc, per_activity, at_boundary)
    pltpu.sync_copy(epart_v, eparts_hbm.at[pl.ds(w * NPAD, NPAD)])
    plsc.subcore_barrier()
    pltpu.sync_copy(
        spacc.at[pl.ds(sid * zw, zw)],
        eparts_hbm.at[pl.ds((NW + cid) * NPAD + sid * zw, zw)])


@functools.partial(
    pl.kernel,
    out_type=jax.ShapeDtypeStruct((NPAD,), _F32),
    mesh=_MESH,
    compiler_params=_PARAMS,
    scratch_types=[
        pltpu.VMEM((NROWS * PW,), _F32),
        pltpu.VMEM((PW,), _F32),
        pltpu.SemaphoreType.DMA,
    ],
)
def _p3w(parts_hbm, out_hbm, rows_v, acc_v, sem):
    w = _wid()
    base = w * PW
    descs = [
        pltpu.async_copy(parts_hbm.at[pl.ds(r * NPAD + base, PW)],
                         rows_v.at[pl.ds(r * PW, PW)], sem)
        for r in range(NROWS)
    ]
    for d in descs:
        d.wait()
    nv = PW // 16

    def body(j, _):
        for u in range(4):
            i = j * 4 + u
            s = rows_v[pl.ds(i * 16, 16)]
            for r in range(1, NROWS):
                s = s + rows_v[pl.ds(r * PW + i * 16, 16)]
            acc_v[pl.ds(i * 16, 16)] = s
        return 0

    lax.fori_loop(0, nv // 4, body, 0)
    pltpu.sync_copy(acc_v, out_hbm.at[pl.ds(base, PW)])


def _p3_body(t):
    @functools.partial(
        pl.kernel,
        out_type=(
            jax.ShapeDtypeStruct((NPAD,), _F32),
            jax.ShapeDtypeStruct((NPAD,), _F32),
            jax.ShapeDtypeStruct((NPAD,), _F32),
        ),
        mesh=_MESH,
        compiler_params=_PARAMS,
        scratch_types=[
            pltpu.VMEM((NROWS * PW,), _F32),
            pltpu.VMEM((PW,), _F32),
            pltpu.VMEM((PW,), _F32),
            pltpu.VMEM((PW,), _F32),
            pltpu.VMEM((PW,), _F32),
            pltpu.VMEM((PW,), _F32),
            pltpu.SemaphoreType.DMA,
        ],
    )
    def _p3(eparts_hbm, wdeg_hbm, trans_hbm, susc_hbm, noise_hbm,
            ni_hbm, trans2_hbm, susc2_hbm,
            rows_v, wdeg_v, tr_v, su_v, no_v, ni_v, sem):
        w = _wid()
        base = w * PW
        descs = [
            pltpu.async_copy(eparts_hbm.at[pl.ds(r * NPAD + base, PW)],
                             rows_v.at[pl.ds(r * PW, PW)], sem)
            for r in range(NROWS)
        ]
        descs.append(pltpu.async_copy(wdeg_hbm.at[pl.ds(base, PW)], wdeg_v, sem))
        descs.append(pltpu.async_copy(trans_hbm.at[pl.ds(base, PW)], tr_v, sem))
        descs.append(pltpu.async_copy(susc_hbm.at[pl.ds(base, PW)], su_v, sem))
        descs.append(pltpu.async_copy(
            noise_hbm.at[pl.ds(t * NPAD + base, PW)], no_v, sem))
        for d in descs:
            d.wait()
        nv = PW // 16

        def body(j, _):
            for u in range(4):
                i = j * 4 + u
                sl = pl.ds(i * 16, 16)
                s = rows_v[sl]
                for r in range(1, NROWS):
                    s = s + rows_v[pl.ds(r * PW + i * 16, 16)]
                expo = s - wdeg_v[sl] * tr_v[sl]
                x = su_v[sl] * expo
                p = 1.0 - jnp.exp(-x)
                p = jnp.clip(p, np.float32(1e-9), np.float32(1.0 - 1e-9))
                q = 1.0 - p
                sg = (q / p) * no_v[sl]
                s2 = sg * sg
                s4 = s2 * s2
                s8 = s4 * s4
                a10 = s8 * s2
                ni = 1.0 / (1.0 + a10)
                ni_v[sl] = ni
                tr_v[sl] = tr_v[sl] + 0.2 * ni
                su_v[sl] = su_v[sl] - ni
            return 0

        lax.fori_loop(0, nv // 4, body, 0)
        pltpu.sync_copy(ni_v, ni_hbm.at[pl.ds(base, PW)])
        pltpu.sync_copy(tr_v, trans2_hbm.at[pl.ds(base, PW)])
        pltpu.sync_copy(su_v, susc2_hbm.at[pl.ds(base, PW)])

    return _p3


_P3 = [_p3_body(t) for t in range(T)]


def kernel(transmissions, susceptibilities, beta_parameters,
           people_school, groups_school,
           people_company, groups_company,
           people_household, groups_household):
    def pack(p, g):
        return jnp.bitwise_or(p.astype(_I32),
                              jnp.left_shift(g.astype(_I32), PBITS))

    edges = jnp.concatenate([
        pack(people_school, groups_school),
        pack(people_company, groups_company),
        pack(people_household, groups_household),
    ])

    trans0 = jnp.pad(transmissions.astype(_F32), (0, NPAD - N))
    susc0 = jnp.pad(susceptibilities.astype(_F32), (0, NPAD - N))
    beta = jnp.pad(beta_parameters.astype(_F32), (0, 16 - 3))

    noise_key = jax.random.key(42)
    wn = []
    for t in range(T):
        u = jax.random.uniform(jax.random.fold_in(noise_key, t), (2, N),
                               minval=1e-9, maxval=1.0)
        wt = jnp.log(u[0]) / jnp.log(u[1])
        wn.append(jnp.pad(wt, (0, NPAD - N), constant_values=1.0))
    noise = jnp.concatenate(wn)

    ones_gsum = jnp.ones((3 * GPAD,), _F32)
    wparts = _p2(edges, ones_gsum, beta)
    wdeg = _p3w(wparts)

    trans, susc = trans0, susc0
    rets = []
    for t in range(T):
        gparts = _p1(trans, edges)
        gsum = _p1b(gparts)
        eparts = _p2(edges, gsum, beta)
        ni, trans, susc = _P3[t](eparts, wdeg, trans, susc, noise)
        rets.append(ni[:N])
    return jnp.stack(rets, axis=0)


# speedup vs baseline: 254.2971x; 1.0590x over previous
import functools

import jax
import jax.numpy as jnp
import numpy as np
from jax import lax
from jax.experimental import pallas as pl
from jax.experimental.pallas import tpu as pltpu
from jax.experimental.pallas import tpu_sc as plsc

N = 100000
G = 10000
E = 1600000
T = 5
NW = 32
NC = 2
NS = 16
EW = E // NW
CHUNK = 2000
NCHUNK = EW // CHUNK
NPAD = 100352
PW = NPAD // NW
GPAD = 10240
NROWS = NW + 2
GSUMW = 3 * GPAD // NW
PBITS = 17
PMASK = (1 << PBITS) - 1
NV = CHUNK // 16
SV = 60
TECV = NV - SV
SE = SV * 16
UT = 5
US = 5

_MESH = plsc.VectorSubcoreMesh(core_axis_name="c", subcore_axis_name="s")
_PARAMS = pltpu.CompilerParams(needs_layout_passes=False)
_F32 = jnp.float32
_I32 = jnp.int32


def _wid():
    return lax.axis_index("s") * NC + lax.axis_index("c")


def _zero_ref(ref, nwords, unroll=16):
    z = jnp.zeros((16,), _F32)
    nv = nwords // 16

    def body(j, _):
        for u in range(unroll):
            ref[pl.ds((j * unroll + u) * 16, 16)] = z
        return 0

    lax.fori_loop(0, nv // unroll, body, 0)
    for r in range(nv - (nv // unroll) * unroll):
        ref[pl.ds(((nv // unroll) * unroll + r) * 16, 16)] = z


def _edge_engine(edges_hbm, w, ebufs, esems, gxs, vls, ssems, spacc,
                 per_activity, at_boundary):
    def eoff(a, c):
        return (a * NW + w) * EW + c * CHUNK

    def issue_e(a, c, b):
        return pltpu.async_copy(
            edges_hbm.at[pl.ds(eoff(a, c), CHUNK)], ebufs[b], esems[b])

    def wait_e(a, c, b):
        pltpu.make_async_copy(
            edges_hbm.at[pl.ds(eoff(a, c), CHUNK)], ebufs[b], esems[b]).wait()

    def issue_s(sl):
        pltpu.async_copy(vls[sl], spacc.at[gxs[sl]], ssems[sl], add=True)

    def wait_s(sl):
        pltpu.make_async_copy(vls[sl], spacc.at[gxs[sl]], ssems[sl]).wait()

    def proc(ebuf, tec_fn, str_fn, slot):
        def bodyT(i, _):
            for u in range(UT):
                tec_fn(ebuf, i * UT + u)
            return 0

        lax.fori_loop(0, TECV // UT, bodyT, 0)
        if slot is None:
            def bodyS0(i, _):
                for u in range(US):
                    tec_fn(ebuf, TECV + i * US + u)
                return 0

            lax.fori_loop(0, SV // US, bodyS0, 0)
        else:
            def bodyS(i, _):
                for u in range(US):
                    so = i * US + u
                    iv, vv = str_fn(ebuf, TECV + so)
                    gxs[slot][pl.ds(so * 16, 16)] = iv
                    vls[slot][pl.ds(so * 16, 16)] = vv
                return 0

            lax.fori_loop(0, SV // US, bodyS, 0)
            issue_s(slot)

    for a in range(3):
        tec_fn, str_fn = per_activity(a)
        issue_e(a, 0, 0)
        issue_e(a, 1, 1)
        wait_e(a, 0, 0)
        proc(ebufs[0], tec_fn, str_fn, None)
        issue_e(a, 2, 0)

        def pair(j, _):
            @pl.when(j >= 1)
            def _():
                wait_s(1)
                wait_s(0)
            c1 = 2 * j + 1
            wait_e(a, c1, 1)
            proc(ebufs[1], tec_fn, str_fn, 1)
            issue_e(a, c1 + 2, 1)
            c2 = 2 * j + 2
            wait_e(a, c2, 0)
            proc(ebufs[0], tec_fn, str_fn, 0)
            issue_e(a, c2 + 2, 0)
            return 0

        lax.fori_loop(0, (NCHUNK - 3) // 2, pair, 0)
        wait_s(1)
        wait_s(0)
        wait_e(a, NCHUNK - 2, 1)
        proc(ebufs[1], tec_fn, str_fn, 1)
        wait_e(a, NCHUNK - 1, 0)
        proc(ebufs[0], tec_fn, str_fn, 0)
        wait_s(1)
        wait_s(0)
        at_boundary(a)


@functools.partial(
    pl.kernel,
    out_type=jax.ShapeDtypeStruct((NROWS * 3 * GPAD,), _F32),
    mesh=_MESH,
    compiler_params=_PARAMS,
    scratch_types=[
        pltpu.VMEM((NPAD,), _F32),
        pltpu.VMEM((GPAD,), _F32),
        pltpu.VMEM((CHUNK,), _I32),
        pltpu.VMEM((CHUNK,), _I32),
        pltpu.VMEM((SE,), _I32),
        pltpu.VMEM((SE,), _I32),
        pltpu.VMEM((SE,), _F32),
        pltpu.VMEM((SE,), _F32),
        pltpu.VMEM_SHARED((3 * GPAD,), _F32),
        pltpu.SemaphoreType.DMA,
        pltpu.SemaphoreType.DMA,
        pltpu.SemaphoreType.DMA,
        pltpu.SemaphoreType.DMA,
        pltpu.SemaphoreType.DMA,
    ],
)
def _p1(trans_hbm, edges_hbm, gparts_hbm, trans_v, gpart_v, eb0, eb1,
        gxa, gxb, vla, vlb, spacc, se0, se1, ss0, ss1, st):
    w = _wid()
    sid = lax.axis_index("s")
    cid = lax.axis_index("c")
    tcopy = pltpu.async_copy(trans_hbm, trans_v, st)
    _zero_ref(gpart_v, GPAD)

    @pl.when(sid == 0)
    def _():
        for a3 in range(3):
            pltpu.sync_copy(gpart_v, spacc.at[pl.ds(a3 * GPAD, GPAD)])

    plsc.subcore_barrier()
    tcopy.wait()

    def per_activity(a):
        goff = a * GPAD

        def tec_fn(ebuf, jv):
            packed = ebuf[pl.ds(jv * 16, 16)]
            p = jnp.bitwise_and(packed, PMASK)
            g = jnp.right_shift(packed, PBITS)
            tv = plsc.load_gather(trans_v, [p])
            plsc.addupdate_scatter(gpart_v, [g], tv)

        def str_fn(ebuf, jv):
            packed = ebuf[pl.ds(jv * 16, 16)]
            p = jnp.bitwise_and(packed, PMASK)
            g = jnp.right_shift(packed, PBITS)
            tv = plsc.load_gather(trans_v, [p])
            return g + goff, tv

        return tec_fn, str_fn

    def at_boundary(a):
        pltpu.sync_copy(gpart_v, gparts_hbm.at[pl.ds((w * 3 + a) * GPAD, GPAD)])
        if a + 1 < 3:
            _zero_ref(gpart_v, GPAD)

    _edge_engine(edges_hbm, w, (eb0, eb1), (se0, se1), (gxa, gxb),
                 (vla, vlb), (ss0, ss1), spacc, per_activity, at_boundary)
    plsc.subcore_barrier()
    sw = 3 * GPAD // NS
    pltpu.sync_copy(
        spacc.at[pl.ds(sid * sw, sw)],
        gparts_hbm.at[pl.ds((NW + cid) * 3 * GPAD + sid * sw, sw)])


@functools.partial(
    pl.kernel,
    out_type=jax.ShapeDtypeStruct((3 * GPAD,), _F32),
    mesh=_MESH,
    compiler_params=_PARAMS,
    scratch_types=[
        pltpu.VMEM((NROWS * GSUMW,), _F32),
        pltpu.VMEM((GSUMW,), _F32),
        pltpu.SemaphoreType.DMA,
    ],
)
def _p1b(gparts_hbm, gsum_hbm, rows_v, acc_v, sem):
    w = _wid()
    descs = [
        pltpu.async_copy(
            gparts_hbm.at[pl.ds(r * 3 * GPAD + w * GSUMW, GSUMW)],
            rows_v.at[pl.ds(r * GSUMW, GSUMW)], sem)
        for r in range(NROWS)
    ]
    for d in descs:
        d.wait()
    nv = GSUMW // 16

    def body(j, _):
        for u in range(4):
            i = j * 4 + u
            s = rows_v[pl.ds(i * 16, 16)]
            for r in range(1, NROWS):
                s = s + rows_v[pl.ds(r * GSUMW + i * 16, 16)]
            acc_v[pl.ds(i * 16, 16)] = s
        return 0

    lax.fori_loop(0, nv // 4, body, 0)
    pltpu.sync_copy(acc_v, gsum_hbm.at[pl.ds(w * GSUMW, GSUMW)])


@functools.partial(
    pl.kernel,
    out_type=jax.ShapeDtypeStruct((NROWS * NPAD,), _F32),
    mesh=_MESH,
    compiler_params=_PARAMS,
    scratch_types=[
        pltpu.VMEM((NPAD,), _F32),
        pltpu.VMEM((GPAD,), _F32),
        pltpu.VMEM((CHUNK,), _I32),
        pltpu.VMEM((CHUNK,), _I32),
        pltpu.VMEM((SE,), _I32),
        pltpu.VMEM((SE,), _I32),
        pltpu.VMEM((SE,), _F32),
        pltpu.VMEM((SE,), _F32),
        pltpu.VMEM((16,), _F32),
        pltpu.VMEM_SHARED((NPAD,), _F32),
        pltpu.SemaphoreType.DMA,
        pltpu.SemaphoreType.DMA,
        pltpu.SemaphoreType.DMA,
        pltpu.SemaphoreType.DMA,
        pltpu.SemaphoreType.DMA,
    ],
)
def _p2(edges_hbm, gsum_hbm, beta_hbm, eparts_hbm, epart_v, gbuf_v,
        eb0, eb1, gxa, gxb, vla, vlb, beta_v, spacc, se0, se1, ss0, ss1, sg):
    w = _wid()
    sid = lax.axis_index("s")
    cid = lax.axis_index("c")
    pltpu.sync_copy(beta_hbm, beta_v)
    gd = pltpu.async_copy(gsum_hbm.at[pl.ds(0, GPAD)], gbuf_v, sg)
    _zero_ref(epart_v, NPAD)
    zw = NPAD // NS
    pltpu.sync_copy(epart_v.at[pl.ds(sid * zw, zw)],
                    spacc.at[pl.ds(sid * zw, zw)])
    plsc.subcore_barrier()

    state = {"gd": gd}

    def per_activity(a):
        state["gd"].wait()
        beta_a = plsc.load_gather(beta_v, [jnp.full((16,), a, _I32)])

        def tec_fn(ebuf, jv):
            packed = ebuf[pl.ds(jv * 16, 16)]
            p = jnp.bitwise_and(packed, PMASK)
            g = jnp.right_shift(packed, PBITS)
            gv = plsc.load_gather(gbuf_v, [g])
            plsc.addupdate_scatter(epart_v, [p], gv * beta_a)

        def str_fn(ebuf, jv):
            packed = ebuf[pl.ds(jv * 16, 16)]
            p = jnp.bitwise_and(packed, PMASK)
            g = jnp.right_shift(packed, PBITS)
            gv = plsc.load_gather(gbuf_v, [g])
            return p, gv * beta_a

        return tec_fn, str_fn

    def at_boundary(a):
        if a + 1 < 3:
            state["gd"] = pltpu.async_copy(
                gsum_hbm.at[pl.ds((a + 1) * GPAD, GPAD)], gbuf_v, sg)

    _edge_engine(edges_hbm, w, (eb0, eb1), (se0, se1), (gxa, gxb),
                 (vla, vlb), (ss0, ss1), spacc, per_activity, at_boundary)
    pltpu.sync_copy(epart_v, eparts_hbm.at[pl.ds(w * NPAD, NPAD)])
    plsc.subcore_barrier()
    pltpu.sync_copy(
        spacc.at[pl.ds(sid * zw, zw)],
        eparts_hbm.at[pl.ds((NW + cid) * NPAD + sid * zw, zw)])


@functools.partial(
    pl.kernel,
    out_type=jax.ShapeDtypeStruct((NROWS * NPAD,), _F32),
    mesh=_MESH,
    compiler_params=_PARAMS,
    scratch_types=[
        pltpu.VMEM((NPAD,), _F32),
        pltpu.VMEM((CHUNK,), _I32),
        pltpu.VMEM((CHUNK,), _I32),
        pltpu.VMEM((SE,), _I32),
        pltpu.VMEM((SE,), _I32),
        pltpu.VMEM((SE,), _F32),
        pltpu.VMEM((SE,), _F32),
        pltpu.VMEM((16,), _F32),
        pltpu.VMEM_SHARED((NPAD,), _F32),
        pltpu.SemaphoreType.DMA,
        pltpu.SemaphoreType.DMA,
        pltpu.SemaphoreType.DMA,
        pltpu.SemaphoreType.DMA,
    ],
)
def _p0(edges_hbm, beta_hbm, wparts_hbm, wpart_v, eb0, eb1,
        gxa, gxb, vla, vlb, beta_v, spacc, se0, se1, ss0, ss1):
    w = _wid()
    sid = lax.axis_index("s")
    cid = lax.axis_index("c")
    pltpu.sync_copy(beta_hbm, beta_v)
    _zero_ref(wpart_v, NPAD)
    zw = NPAD // NS
    pltpu.sync_copy(wpart_v.at[pl.ds(sid * zw, zw)],
                    spacc.at[pl.ds(sid * zw, zw)])
    plsc.subcore_barrier()

    def per_activity(a):
        beta_a = plsc.load_gather(beta_v, [jnp.full((16,), a, _I32)])

        def tec_fn(ebuf, jv):
            packed = ebuf[pl.ds(jv * 16, 16)]
            p = jnp.bitwise_and(packed, PMASK)
            plsc.addupdate_scatter(wpart_v, [p], beta_a)

        def str_fn(ebuf, jv):
            packed = ebuf[pl.ds(jv * 16, 16)]
            p = jnp.bitwise_and(packed, PMASK)
            return p, beta_a

        return tec_fn, str_fn

    def at_boundary(a):
        pass

    _edge_engine(edges_hbm, w, (eb0, eb1), (se0, se1), (gxa, gxb),
                 (vla, vlb), (ss0, ss1), spacc, per_activity, at_boundary)
    pltpu.sync_copy(wpart_v, wparts_hbm.at[pl.ds(w * NPAD, NPAD)])
    plsc.subcore_barrier()
    pltpu.sync_copy(
        spacc.at[pl.ds(sid * zw, zw)],
        wparts_hbm.at[pl.ds((NW + cid) * NPAD + sid * zw, zw)])


@functools.partial(
    pl.kernel,
    out_type=jax.ShapeDtypeStruct((NPAD,), _F32),
    mesh=_MESH,
    compiler_params=_PARAMS,
    scratch_types=[
        pltpu.VMEM((NROWS * PW,), _F32),
        pltpu.VMEM((PW,), _F32),
        pltpu.SemaphoreType.DMA,
    ],
)
def _p3w(parts_hbm, out_hbm, rows_v, acc_v, sem):
    w = _wid()
    base = w * PW
    descs = [
        pltpu.async_copy(parts_hbm.at[pl.ds(r * NPAD + base, PW)],
                         rows_v.at[pl.ds(r * PW, PW)], sem)
        for r in range(NROWS)
    ]
    for d in descs:
        d.wait()
    nv = PW // 16

    def body(j, _):
        for u in range(4):
            i = j * 4 + u
            s = rows_v[pl.ds(i * 16, 16)]
            for r in range(1, NROWS):
                s = s + rows_v[pl.ds(r * PW + i * 16, 16)]
            acc_v[pl.ds(i * 16, 16)] = s
        return 0

    lax.fori_loop(0, nv // 4, body, 0)
    pltpu.sync_copy(acc_v, out_hbm.at[pl.ds(base, PW)])


def _p3_body(t):
    @functools.partial(
        pl.kernel,
        out_type=(
            jax.ShapeDtypeStruct((NPAD,), _F32),
            jax.ShapeDtypeStruct((NPAD,), _F32),
            jax.ShapeDtypeStruct((NPAD,), _F32),
        ),
        mesh=_MESH,
        compiler_params=_PARAMS,
        scratch_types=[
            pltpu.VMEM((NROWS * PW,), _F32),
            pltpu.VMEM((PW,), _F32),
            pltpu.VMEM((PW,), _F32),
            pltpu.VMEM((PW,), _F32),
            pltpu.VMEM((PW,), _F32),
            pltpu.VMEM((PW,), _F32),
            pltpu.SemaphoreType.DMA,
        ],
    )
    def _p3(eparts_hbm, wdeg_hbm, trans_hbm, susc_hbm, noise_hbm,
            ni_hbm, trans2_hbm, susc2_hbm,
            rows_v, wdeg_v, tr_v, su_v, no_v, ni_v, sem):
        w = _wid()
        base = w * PW
        descs = [
            pltpu.async_copy(eparts_hbm.at[pl.ds(r * NPAD + base, PW)],
                             rows_v.at[pl.ds(r * PW, PW)], sem)
            for r in range(NROWS)
        ]
        descs.append(pltpu.async_copy(wdeg_hbm.at[pl.ds(base, PW)], wdeg_v, sem))
        descs.append(pltpu.async_copy(trans_hbm.at[pl.ds(base, PW)], tr_v, sem))
        descs.append(pltpu.async_copy(susc_hbm.at[pl.ds(base, PW)], su_v, sem))
        descs.append(pltpu.async_copy(
            noise_hbm.at[pl.ds(t * NPAD + base, PW)], no_v, sem))
        for d in descs:
            d.wait()
        nv = PW // 16

        def body(j, _):
            for u in range(4):
                i = j * 4 + u
                sl = pl.ds(i * 16, 16)
                s = rows_v[sl]
                for r in range(1, NROWS):
                    s = s + rows_v[pl.ds(r * PW + i * 16, 16)]
                expo = s - wdeg_v[sl] * tr_v[sl]
                x = su_v[sl] * expo
                p = 1.0 - jnp.exp(-x)
                p = jnp.clip(p, np.float32(1e-9), np.float32(1.0 - 1e-9))
                q = 1.0 - p
                sg = (q / p) * no_v[sl]
                s2 = sg * sg
                s4 = s2 * s2
                s8 = s4 * s4
                a10 = s8 * s2
                ni = 1.0 / (1.0 + a10)
                ni_v[sl] = ni
                tr_v[sl] = tr_v[sl] + 0.2 * ni
                su_v[sl] = su_v[sl] - ni
            return 0

        lax.fori_loop(0, nv // 4, body, 0)
        pltpu.sync_copy(ni_v, ni_hbm.at[pl.ds(base, PW)])
        pltpu.sync_copy(tr_v, trans2_hbm.at[pl.ds(base, PW)])
        pltpu.sync_copy(su_v, susc2_hbm.at[pl.ds(base, PW)])

    return _p3


_P3 = [_p3_body(t) for t in range(T)]


def kernel(transmissions, susceptibilities, beta_parameters,
           people_school, groups_school,
           people_company, groups_company,
           people_household, groups_household):
    def pack(p, g):
        return jnp.bitwise_or(p.astype(_I32),
                              jnp.left_shift(g.astype(_I32), PBITS))

    edges = jnp.concatenate([
        pack(people_school, groups_school),
        pack(people_company, groups_company),
        pack(people_household, groups_household),
    ])

    trans0 = jnp.pad(transmissions.astype(_F32), (0, NPAD - N))
    susc0 = jnp.pad(susceptibilities.astype(_F32), (0, NPAD - N))
    beta = jnp.pad(beta_parameters.astype(_F32), (0, 16 - 3))

    noise_key = jax.random.key(42)
    wn = []
    for t in range(T):
        u = jax.random.uniform(jax.random.fold_in(noise_key, t), (2, N),
                               minval=1e-9, maxval=1.0)
        wt = jnp.log(u[0]) / jnp.log(u[1])
        wn.append(jnp.pad(wt, (0, NPAD - N), constant_values=1.0))
    noise = jnp.concatenate(wn)

    wparts = _p0(edges, beta)
    wdeg = _p3w(wparts)

    trans, susc = trans0, susc0
    rets = []
    for t in range(T):
        gparts = _p1(trans, edges)
        gsum = _p1b(gparts)
        eparts = _p2(edges, gsum, beta)
        ni, trans, susc = _P3[t](eparts, wdeg, trans, susc, noise)
        rets.append(ni[:N])
    return jnp.stack(rets, axis=0)
